# Initial kernel scaffold; baseline (speedup 1.0000x reference)
import jax
import jax.numpy as jnp
from jax.experimental import pallas as pl


def kernel(S, u):
    raise NotImplementedError("write your pallas kernel here")


# speedup vs baseline: 12.1824x; 12.1824x over previous
import jax
import jax.numpy as jnp
from jax.experimental import pallas as pl

_K = 64
_R = 64
_C = 8192


def _body(s_ref, u_ref, o_ref):
    S = s_ref[...]
    u = u_ref[...]

    m = jnp.max(S, axis=1, keepdims=True)
    e = jnp.exp(S - m)
    denom = jnp.sum(e, axis=1, keepdims=True)
    Sn = e / denom

    g = -jnp.log(-jnp.log(u + 1e-10) + 1e-10)
    P = Sn + g
    P = jnp.where(P == 0.0, 0.0, P)

    b = jax.lax.bitcast_convert_type(P, jnp.uint32)
    neg = b >= jnp.uint32(0x80000000)
    key = jnp.where(neg, ~b, b | jnp.uint32(0x80000000))

    def step(i, T):
        bit = jnp.uint32(1) << (jnp.uint32(31) - i.astype(jnp.uint32))
        cand = T | bit
        cnt = jnp.sum((key >= cand).astype(jnp.int32), axis=1, keepdims=True)
        return jnp.where(cnt >= _K, cand, T)

    T = jax.lax.fori_loop(
        0, 32, step, jnp.zeros((_R, 1), jnp.uint32), unroll=True)

    gt = key > T
    eq = key == T
    need = _K - jnp.sum(gt.astype(jnp.int32), axis=1, keepdims=True)

    idx = jax.lax.broadcasted_iota(jnp.int32, (_R, _C), 1)

    def jstep(i, J):
        bit = jnp.int32(1) << (jnp.int32(12) - i)
        cand = J | bit
        cnt = jnp.sum((eq & (idx < cand)).astype(jnp.int32), axis=1,
                      keepdims=True)
        return jnp.where(cnt < need, cand, J)

    J = jax.lax.fori_loop(
        0, 13, jstep, jnp.zeros((_R, 1), jnp.int32), unroll=True)

    mask = gt | (eq & (idx <= J))
    o_ref[...] = jnp.where(mask, Sn, 0.0)


def kernel(S, u):
    return pl.pallas_call(
        _body,
        out_shape=jax.ShapeDtypeStruct((_R, _C), jnp.float32),
    )(S, u)
